# baseline (speedup 1.0000x reference)
import functools

import jax
import jax.numpy as jnp
import numpy as np
from jax import lax
from jax.experimental import pallas as pl
from jax.experimental.pallas import tpu as pltpu
from jax.experimental.pallas import tpu_sc as plsc

NUM_CLASSES = 1000
B0 = 4096
B1 = 26
L = 16
NC = 2
NW = 32
D0_PER_W = B0 // NW
N_CH = D0_PER_W
XE_W = 2 * L

_CONSTS = np.concatenate([
    np.zeros(L, np.int32),
    np.ones(L, np.int32),
    np.arange(L, dtype=np.int32),
    np.concatenate([np.arange(L, B1, dtype=np.int32),
                    np.arange(2 * L - B1, dtype=np.int32)]),
])


def _sc_onehot(xe_hbm, c_hbm, z_hbm, o_hbm, buf_a, buf_b, xe_l, c_l,
               sem_a, sem_b):
    wid = lax.axis_index("s") * NC + lax.axis_index("c")
    base_d = wid * D0_PER_W

    pltpu.sync_copy(xe_hbm.at[pl.ds(base_d * XE_W, D0_PER_W * XE_W)], xe_l)
    pltpu.sync_copy(c_hbm, c_l)
    pltpu.sync_copy(z_hbm, buf_a)
    pltpu.sync_copy(z_hbm, buf_b)

    def scatter_val(buf, c, val_slot):
        vals = c_l[pl.ds(val_slot * L, L)]
        d0v = c_l[pl.ds(0, L)]
        for v in range(2):
            d1v = c_l[pl.ds((2 + v) * L, L)]
            cols = xe_l[pl.ds(c * XE_W + v * L, L)]
            plsc.store_scatter(buf, [d0v, d1v, cols], vals)

    def chunk(c, buf, sem):
        d0 = base_d + c

        @pl.when(c >= 2)
        def _wait_and_clear():
            pltpu.make_async_copy(
                buf, o_hbm.at[pl.ds(d0 - 2, 1)], sem
            ).wait()
            scatter_val(buf, c - 2, 0)

        scatter_val(buf, c, 1)
        pltpu.async_copy(buf, o_hbm.at[pl.ds(d0, 1)], sem)

    def outer(i, carry):
        chunk(2 * i, buf_a, sem_a)
        chunk(2 * i + 1, buf_b, sem_b)
        return carry

    lax.fori_loop(0, N_CH // 2, outer, 0)
    pltpu.make_async_copy(
        buf_a, o_hbm.at[pl.ds(base_d + N_CH - 2, 1)], sem_a
    ).wait()
    pltpu.make_async_copy(
        buf_b, o_hbm.at[pl.ds(base_d + N_CH - 1, 1)], sem_b
    ).wait()


@jax.jit
def _onehot_sc(x):
    xe = jnp.concatenate([x, x[:, : XE_W - B1]], axis=1).reshape(B0 * XE_W)
    consts = jnp.asarray(_CONSTS)
    z = jnp.zeros((1, B1, NUM_CLASSES), jnp.int32)
    run = functools.partial(
        pl.kernel,
        mesh=plsc.VectorSubcoreMesh(core_axis_name="c", subcore_axis_name="s"),
        compiler_params=pltpu.CompilerParams(
            use_tc_tiling_on_sc=False, needs_layout_passes=False
        ),
        out_type=jax.ShapeDtypeStruct((B0, B1, NUM_CLASSES), jnp.int32),
        scratch_types=[
            pltpu.VMEM((1, B1, NUM_CLASSES), jnp.int32),
            pltpu.VMEM((1, B1, NUM_CLASSES), jnp.int32),
            pltpu.VMEM((D0_PER_W * XE_W,), jnp.int32),
            pltpu.VMEM((len(_CONSTS),), jnp.int32),
            pltpu.SemaphoreType.DMA,
            pltpu.SemaphoreType.DMA,
        ],
    )(_sc_onehot)
    return run(xe, consts, z)


def kernel(x):
    return _onehot_sc(x)


# speedup vs baseline: 1.9720x; 1.9720x over previous
import functools

import jax
import jax.numpy as jnp
import numpy as np
from jax import lax
from jax.experimental import pallas as pl
from jax.experimental.pallas import tpu as pltpu
from jax.experimental.pallas import tpu_sc as plsc

NUM_CLASSES = 1000
B0 = 4096
B1 = 26
L = 16
NC = 2
NW = 32
D0_PER_W = B0 // NW
N_CH = D0_PER_W
XE_W = 2 * L

_CONSTS = np.concatenate([
    np.zeros(L, np.int32),
    np.ones(L, np.int32),
    np.arange(L, dtype=np.int32),
    np.concatenate([np.arange(L, B1, dtype=np.int32),
                    np.arange(2 * L - B1, dtype=np.int32)]),
])


def _sc_onehot(xe_hbm, c_hbm, z_hbm, o_hbm, buf_a, buf_b, xe_l, c_l,
               sem_a, sem_b):
    wid = lax.axis_index("s") * NC + lax.axis_index("c")
    base_d = wid * D0_PER_W

    pltpu.sync_copy(xe_hbm.at[pl.ds(base_d * XE_W, D0_PER_W * XE_W)], xe_l)
    pltpu.sync_copy(c_hbm, c_l)
    pltpu.sync_copy(z_hbm, buf_a)
    pltpu.sync_copy(z_hbm, buf_b)

    def scatter_val(buf, c, val_slot):
        vals = c_l[pl.ds(val_slot * L, L)]
        d0v = c_l[pl.ds(0, L)]
        for v in range(2):
            d1v = c_l[pl.ds((2 + v) * L, L)]
            cols = xe_l[pl.ds(c * XE_W + v * L, L)]
            plsc.store_scatter(buf, [d0v, d1v, cols], vals)

    def chunk(c, buf, sem):
        d0 = base_d + c

        @pl.when(c >= 2)
        def _wait_and_clear():
            pltpu.make_async_copy(
                buf, o_hbm.at[pl.ds(d0 - 2, 1)], sem
            ).wait()
            scatter_val(buf, c - 2, 0)

        scatter_val(buf, c, 1)
        pltpu.async_copy(buf, o_hbm.at[pl.ds(d0, 1)], sem)

    def outer(i, carry):
        chunk(2 * i, buf_a, sem_a)
        chunk(2 * i + 1, buf_b, sem_b)
        return carry

    lax.fori_loop(0, N_CH // 2, outer, 0)
    pltpu.make_async_copy(
        buf_a, o_hbm.at[pl.ds(base_d + N_CH - 2, 1)], sem_a
    ).wait()
    pltpu.make_async_copy(
        buf_b, o_hbm.at[pl.ds(base_d + N_CH - 1, 1)], sem_b
    ).wait()


@jax.jit
def _onehot_sc(x):
    xe = jnp.concatenate([x, x[:, : XE_W - B1]], axis=1).reshape(B0 * XE_W)
    consts = jnp.asarray(_CONSTS)
    z = jnp.zeros((1, B1, NUM_CLASSES), jnp.int32)
    run = functools.partial(
        pl.kernel,
        mesh=plsc.VectorSubcoreMesh(core_axis_name="c", subcore_axis_name="s"),
        compiler_params=pltpu.CompilerParams(
            use_tc_tiling_on_sc=True, needs_layout_passes=False
        ),
        out_type=jax.ShapeDtypeStruct((B0, B1, NUM_CLASSES), jnp.int32),
        scratch_types=[
            pltpu.VMEM((1, B1, NUM_CLASSES), jnp.int32),
            pltpu.VMEM((1, B1, NUM_CLASSES), jnp.int32),
            pltpu.VMEM((D0_PER_W * XE_W,), jnp.int32),
            pltpu.VMEM((len(_CONSTS),), jnp.int32),
            pltpu.SemaphoreType.DMA,
            pltpu.SemaphoreType.DMA,
        ],
    )(_sc_onehot)
    return run(xe, consts, z)


def kernel(x):
    return _onehot_sc(x)


# speedup vs baseline: 7.0306x; 3.5653x over previous
import functools

import jax
import jax.numpy as jnp
import numpy as np
from jax import lax
from jax.experimental import pallas as pl
from jax.experimental.pallas import tpu as pltpu
from jax.experimental.pallas import tpu_sc as plsc

NUM_CLASSES = 1000
B0 = 4096
B1 = 26
L = 16
NC = 2
NW = 32
D0_PER_W = B0 // NW
N_VEC = D0_PER_W // L
X_PER_W = D0_PER_W * B1

_CONSTS = np.concatenate([
    np.zeros(L, np.int32),
    np.ones(L, np.int32),
    np.arange(D0_PER_W, dtype=np.int32),
])


def _sc_onehot(xt_hbm, c_hbm, z_hbm, o_hbm, buf, xt_l, c_l, sem):
    wid = lax.axis_index("s") * NC + lax.axis_index("c")

    pltpu.sync_copy(c_hbm, c_l)
    pltpu.sync_copy(z_hbm, buf)

    def scatter_val(val_slot):
        vals = c_l[pl.ds(val_slot * L, L)]
        d1v = c_l[pl.ds(0, L)]
        for v in range(N_VEC):
            d0v = c_l[pl.ds(2 * L + v * L, L)]
            cv = xt_l[pl.ds(v * L, L)]
            plsc.store_scatter(buf, [d1v, cv, d0v], vals)

    def dst(d1):
        return o_hbm.at[
            pl.ds(d1, 1), pl.ds(0, NUM_CLASSES), pl.ds(wid * D0_PER_W, D0_PER_W)
        ]

    def chunk(d1, carry):
        @pl.when(d1 >= 1)
        def _wait_and_clear():
            pltpu.make_async_copy(buf, dst(d1 - 1), sem).wait()
            scatter_val(0)

        pltpu.sync_copy(
            xt_hbm.at[pl.ds(wid * X_PER_W + d1 * D0_PER_W, D0_PER_W)], xt_l
        )
        scatter_val(1)
        pltpu.async_copy(buf, dst(d1), sem)
        return carry

    lax.fori_loop(0, B1, chunk, 0)
    pltpu.make_async_copy(buf, dst(B1 - 1), sem).wait()


@jax.jit
def _onehot_sc(x):
    xt = x.reshape(NW, D0_PER_W, B1).transpose(0, 2, 1).reshape(NW * X_PER_W)
    consts = jnp.asarray(_CONSTS)
    z = jnp.zeros((1, NUM_CLASSES, D0_PER_W), jnp.int32)
    run = functools.partial(
        pl.kernel,
        mesh=plsc.VectorSubcoreMesh(core_axis_name="c", subcore_axis_name="s"),
        compiler_params=pltpu.CompilerParams(
            use_tc_tiling_on_sc=True, needs_layout_passes=False
        ),
        out_type=jax.ShapeDtypeStruct((B1, NUM_CLASSES, B0), jnp.int32),
        scratch_types=[
            pltpu.VMEM((1, NUM_CLASSES, D0_PER_W), jnp.int32),
            pltpu.VMEM((D0_PER_W,), jnp.int32),
            pltpu.VMEM((len(_CONSTS),), jnp.int32),
            pltpu.SemaphoreType.DMA,
        ],
    )(_sc_onehot)
    out = run(xt, consts, z)
    return jnp.transpose(out, (2, 0, 1))


def kernel(x):
    return _onehot_sc(x)


# speedup vs baseline: 7.5274x; 1.0707x over previous
import functools

import jax
import jax.numpy as jnp
import numpy as np
from jax import lax
from jax.experimental import pallas as pl
from jax.experimental.pallas import tpu as pltpu
from jax.experimental.pallas import tpu_sc as plsc

NUM_CLASSES = 1000
B0 = 4096
B1 = 26
L = 16
NC = 2
NW = 32
D0_PER_W = B0 // NW
N_VEC = D0_PER_W // L
X_PER_W = D0_PER_W * B1

_CONSTS = np.concatenate([
    np.zeros(L, np.int32),
    np.ones(L, np.int32),
    np.arange(D0_PER_W, dtype=np.int32),
])


def _sc_onehot(xt_hbm, c_hbm, z_hbm, o_hbm, buf, xt_a, xt_b, c_l, sem):
    wid = lax.axis_index("s") * NC + lax.axis_index("c")
    xt_base = wid * X_PER_W

    pltpu.sync_copy(c_hbm, c_l)
    pltpu.sync_copy(z_hbm, buf)
    pltpu.sync_copy(xt_hbm.at[pl.ds(xt_base, D0_PER_W)], xt_a)

    def scatter_val(xt_l, val_slot):
        vals = c_l[pl.ds(val_slot * L, L)]
        d1v = c_l[pl.ds(0, L)]
        for v in range(N_VEC):
            d0v = c_l[pl.ds(2 * L + v * L, L)]
            cv = xt_l[pl.ds(v * L, L)]
            plsc.store_scatter(buf, [d1v, cv, d0v], vals)

    def dst(d1):
        return o_hbm.at[
            pl.ds(d1, 1), pl.ds(0, NUM_CLASSES), pl.ds(wid * D0_PER_W, D0_PER_W)
        ]

    def step(d1, xt_cur, xt_other):
        @pl.when(d1 >= 1)
        def _wait_and_clear():
            pltpu.make_async_copy(buf, dst(d1 - 1), sem).wait()
            scatter_val(xt_other, 0)

        scatter_val(xt_cur, 1)
        pltpu.async_copy(buf, dst(d1), sem)

        @pl.when(d1 < B1 - 1)
        def _prefetch():
            pltpu.sync_copy(
                xt_hbm.at[pl.ds(xt_base + (d1 + 1) * D0_PER_W, D0_PER_W)],
                xt_other,
            )

    def pair(i, carry):
        step(2 * i, xt_a, xt_b)
        step(2 * i + 1, xt_b, xt_a)
        return carry

    lax.fori_loop(0, B1 // 2, pair, 0)
    pltpu.make_async_copy(buf, dst(B1 - 1), sem).wait()


@jax.jit
def _onehot_sc(x):
    xt = x.reshape(NW, D0_PER_W, B1).transpose(0, 2, 1).reshape(NW * X_PER_W)
    consts = jnp.asarray(_CONSTS)
    z = jnp.zeros((1, NUM_CLASSES, D0_PER_W), jnp.int32)
    run = functools.partial(
        pl.kernel,
        mesh=plsc.VectorSubcoreMesh(core_axis_name="c", subcore_axis_name="s"),
        compiler_params=pltpu.CompilerParams(
            use_tc_tiling_on_sc=True, needs_layout_passes=False
        ),
        out_type=jax.ShapeDtypeStruct((B1, NUM_CLASSES, B0), jnp.int32),
        scratch_types=[
            pltpu.VMEM((1, NUM_CLASSES, D0_PER_W), jnp.int32),
            pltpu.VMEM((D0_PER_W,), jnp.int32),
            pltpu.VMEM((D0_PER_W,), jnp.int32),
            pltpu.VMEM((len(_CONSTS),), jnp.int32),
            pltpu.SemaphoreType.DMA,
        ],
    )(_sc_onehot)
    out = run(xt, consts, z)
    return jnp.transpose(out, (2, 0, 1))


def kernel(x):
    return _onehot_sc(x)
